# baseline (device time: 246098 ns/iter reference)
import os

import jax
import jax.numpy as jnp
from jax import lax
from jax.experimental import pallas as pl
from jax.experimental.pallas import tpu as pltpu

_PHASES = os.environ.get("SCB_KERNEL_PHASES", "full")

NZ = 4
M = 8192
D = 2048
QROWS = M // 4
CHUNK = QROWS // NZ
EPS = 1e-6
F32 = jnp.float32
BF16 = jnp.bfloat16

HC = CHUNK // 2

_RS = 0
_AG = 6
_AX = 9
_AY = 13
_BX = 17
_BY = 19


def kernel(partial, resid, gamma):
    gamma2 = gamma.reshape(1, D)

    def body(part_ref, resid_ref, gamma_ref, out_ref,
             pc, send0, rs_recv, q_buf, qx_recv, qy_recv, qd_buf,
             resid_chunk, send_sems, recv_sems, local_sems, store_sems):
        x = lax.axis_index("x")
        y = lax.axis_index("y")
        r = lax.axis_index("z")
        right = (x, y, (r + 1) % NZ)
        left = (x, y, (r + NZ - 1) % NZ)
        xn = (1 - x, y, r)
        yn = (x, 1 - y, r)

        q = 2 * x + y
        base = q * QROWS

        barrier = pltpu.get_barrier_semaphore()
        for nbr in (left, right, xn, yn):
            pl.semaphore_signal(
                barrier, inc=1, device_id=nbr,
                device_id_type=pl.DeviceIdType.MESH,
            )
        pl.semaphore_wait(barrier, 4)

        def pchunk_copy(c, slot):
            cp = pltpu.make_async_copy(
                part_ref.at[0, pl.ds(base + c * CHUNK, CHUNK), :],
                pc.at[slot], local_sems.at[slot])
            cp.start()
            return cp

        l0 = pchunk_copy((r + NZ - 1) % NZ, 0)
        l1 = pchunk_copy((r + NZ - 2) % NZ, 1)
        cr = pltpu.make_async_copy(
            resid_ref.at[pl.ds(base + r * CHUNK, CHUNK), :],
            resid_chunk, local_sems.at[2])
        cr.start()

        l0.wait()
        send0[:, :] = pc[0].astype(BF16)
        l2 = pchunk_copy((r + NZ - 3) % NZ, 0)

        def mk_rs(s, h):
            rows = pl.ds(h * HC, HC)
            src = (send0.at[rows, :] if s == 0
                   else rs_recv.at[s - 1, rows, :])
            rdma = pltpu.make_async_remote_copy(
                src_ref=src,
                dst_ref=rs_recv.at[s, rows, :],
                send_sem=send_sems.at[_RS + 2 * s + h],
                recv_sem=recv_sems.at[_RS + 2 * s + h],
                device_id=right,
                device_id_type=pl.DeviceIdType.MESH,
            )
            rdma.start()
            return rdma

        rs = [[None, None] for _ in range(NZ - 1)]
        rs[0][0] = mk_rs(0, 0)
        rs[0][1] = mk_rs(0, 1)
        l1.wait()
        l3 = None
        for s in range(NZ - 1):
            for h in range(2):
                with jax.named_scope(f"rs#s={s}_h={h}"):
                    if s == 1 and h == 0:
                        l2.wait()
                    rs[s][h].wait()
                    if s < NZ - 2:
                        slot = 1 - s
                        rows = slice(h * HC, (h + 1) * HC)
                        acc = (rs_recv[s, rows, :].astype(F32)
                               + pc[slot, rows, :])
                        rs_recv[s, rows, :] = acc.astype(BF16)
                        rs[s + 1][h] = mk_rs(s + 1, h)
                        if s == 0 and h == 1:
                            l3 = pchunk_copy(r, 1)

        with jax.named_scope("ln"):
            l3.wait()
            cr.wait()
            yv = (rs_recv[NZ - 2, :, :].astype(F32) + pc[1]
                  + resid_chunk[:, :])
            ms = jnp.mean(yv * yv, axis=-1, keepdims=True)
            outv = yv * lax.rsqrt(ms + EPS) * gamma_ref[:, :]
            q_buf[pl.ds(r * CHUNK, CHUNK), :] = outv.astype(BF16)

        stores = []

        def store(src_slice, row_start):
            cp = pltpu.make_async_copy(
                src_slice, out_ref.at[pl.ds(row_start, CHUNK), :],
                store_sems.at[len(stores)])
            cp.start()
            stores.append(cp)

        def abs_k(t):
            return (r + NZ - t) % NZ

        ax_list, ay_list = [], []

        def start_a(t):
            k = abs_k(t)
            sl = (pl.ds(k * CHUNK, CHUNK), slice(None))
            a = pltpu.make_async_remote_copy(
                src_ref=q_buf.at[sl], dst_ref=qx_recv.at[sl],
                send_sem=send_sems.at[_AX + t],
                recv_sem=recv_sems.at[_AX + t],
                device_id=xn, device_id_type=pl.DeviceIdType.MESH,
            )
            a.start()
            ax_list.append(a)
            a = pltpu.make_async_remote_copy(
                src_ref=q_buf.at[sl], dst_ref=qy_recv.at[sl],
                send_sem=send_sems.at[_AY + t],
                recv_sem=recv_sems.at[_AY + t],
                device_id=yn, device_id_type=pl.DeviceIdType.MESH,
            )
            a.start()
            ay_list.append(a)

        def mk_ag(t):
            sl = (pl.ds(abs_k(t) * CHUNK, CHUNK), slice(None))
            rdma = pltpu.make_async_remote_copy(
                src_ref=q_buf.at[sl], dst_ref=q_buf.at[sl],
                send_sem=send_sems.at[_AG + t],
                recv_sem=recv_sems.at[_AG + t],
                device_id=right, device_id_type=pl.DeviceIdType.MESH,
            )
            rdma.start()
            return rdma

        do_ag = _PHASES in ("ag", "full")
        do_xy = _PHASES == "full"
        if not do_xy:
            start_a = lambda t: None

        if do_ag:
            ag = mk_ag(0)
        start_a(0)
        store(q_buf.at[pl.ds(r * CHUNK, CHUNK), :], base + r * CHUNK)
        for t in range(NZ - 1) if do_ag else ():
            with jax.named_scope(f"ag#t={t}"):
                ag.wait()
                if t < NZ - 2:
                    ag = mk_ag(t + 1)
                k_in = abs_k(t + 1)
                start_a(t + 1)
                store(q_buf.at[pl.ds(k_in * CHUNK, CHUNK), :],
                      base + k_in * CHUNK)

        qx = 2 * (1 - x) + y
        qy = 2 * x + (1 - y)
        qd = 2 * (1 - x) + (1 - y)
        b_list = []
        for t in range(NZ) if do_xy else ():
          with jax.named_scope(f"xy#t={t}"):
            k = abs_k(t)
            sl = (pl.ds(k * CHUNK, CHUNK), slice(None))
            if t < 2:
                ay_list[t].wait()
                b = pltpu.make_async_remote_copy(
                    src_ref=qy_recv.at[sl], dst_ref=qd_buf.at[sl],
                    send_sem=send_sems.at[_BX + t],
                    recv_sem=recv_sems.at[_BX + t],
                    device_id=xn, device_id_type=pl.DeviceIdType.MESH,
                )
                b.start()
                ax_list[t].wait()
            else:
                ax_list[t].wait()
                b = pltpu.make_async_remote_copy(
                    src_ref=qx_recv.at[sl], dst_ref=qd_buf.at[sl],
                    send_sem=send_sems.at[_BY + (t - 2)],
                    recv_sem=recv_sems.at[_BY + (t - 2)],
                    device_id=yn, device_id_type=pl.DeviceIdType.MESH,
                )
                b.start()
                ay_list[t].wait()
            b_list.append(b)
            store(qx_recv.at[sl], qx * QROWS + k * CHUNK)
            store(qy_recv.at[sl], qy * QROWS + k * CHUNK)

        for t in range(NZ) if do_xy else ():
            with jax.named_scope(f"brelay#t={t}"):
                b_list[t].wait()
                k = abs_k(t)
                store(qd_buf.at[pl.ds(k * CHUNK, CHUNK), :],
                      qd * QROWS + k * CHUNK)

        with jax.named_scope("drain"):
            for cp in stores:
                cp.wait()

    out_shape = jax.ShapeDtypeStruct((M, D), BF16)
    return pl.pallas_call(
        body,
        out_shape=out_shape,
        in_specs=[
            pl.BlockSpec(memory_space=pl.ANY),
            pl.BlockSpec(memory_space=pl.ANY),
            pl.BlockSpec(memory_space=pltpu.VMEM),
        ],
        out_specs=pl.BlockSpec(memory_space=pl.ANY),
        scratch_shapes=[
            pltpu.VMEM((2, CHUNK, D), F32),
            pltpu.VMEM((CHUNK, D), BF16),
            pltpu.VMEM((NZ - 1, CHUNK, D), BF16),
            pltpu.VMEM((QROWS, D), BF16),
            pltpu.VMEM((QROWS, D), BF16),
            pltpu.VMEM((QROWS, D), BF16),
            pltpu.VMEM((QROWS, D), BF16),
            pltpu.VMEM((CHUNK, D), F32),
            pltpu.SemaphoreType.DMA((21,)),
            pltpu.SemaphoreType.DMA((21,)),
            pltpu.SemaphoreType.DMA((3,)),
            pltpu.SemaphoreType.DMA((16,)),
        ],
        compiler_params=pltpu.CompilerParams(
            collective_id=0,
            vmem_limit_bytes=100 * 1024 * 1024,
        ),
    )(partial, resid, gamma2)


# device time: 245189 ns/iter; 1.0037x vs baseline; 1.0037x over previous
import os

import jax
import jax.numpy as jnp
from jax import lax
from jax.experimental import pallas as pl
from jax.experimental.pallas import tpu as pltpu

_PHASES = os.environ.get("SCB_KERNEL_PHASES", "full")

NZ = 4
M = 8192
D = 2048
QROWS = M // 4
CHUNK = QROWS // NZ
EPS = 1e-6
F32 = jnp.float32
BF16 = jnp.bfloat16

HC = CHUNK // 2

_RS = 0
_AG = 6
_AX = 9
_AY = 13
_BX = 17
_BY = 19


def kernel(partial, resid, gamma):
    gamma2 = gamma.reshape(1, D)

    def body(part_ref, resid_ref, gamma_ref, out_ref,
             pc, send0, rs_recv, q_buf, qx_recv, qy_recv, qd_buf,
             resid_chunk, send_sems, recv_sems, local_sems, store_sems):
        x = lax.axis_index("x")
        y = lax.axis_index("y")
        r = lax.axis_index("z")
        right = (x, y, (r + 1) % NZ)
        left = (x, y, (r + NZ - 1) % NZ)
        xn = (1 - x, y, r)
        yn = (x, 1 - y, r)

        q = 2 * x + y
        base = q * QROWS

        barrier = pltpu.get_barrier_semaphore()
        for nbr in (left, right, xn, yn):
            pl.semaphore_signal(
                barrier, inc=1, device_id=nbr,
                device_id_type=pl.DeviceIdType.MESH,
            )
        pl.semaphore_wait(barrier, 4)

        def pchunk_copy(c, slot):
            cp = pltpu.make_async_copy(
                part_ref.at[0, pl.ds(base + c * CHUNK, CHUNK), :],
                pc.at[slot], local_sems.at[slot])
            cp.start()
            return cp

        l0 = pchunk_copy((r + NZ - 1) % NZ, 0)
        l1 = pchunk_copy((r + NZ - 2) % NZ, 1)
        cr = pltpu.make_async_copy(
            resid_ref.at[pl.ds(base + r * CHUNK, CHUNK), :],
            resid_chunk, local_sems.at[2])
        cr.start()

        l0.wait()

        def mk_rs(s, h):
            rows = pl.ds(h * HC, HC)
            src = (send0.at[rows, :] if s == 0
                   else rs_recv.at[s - 1, rows, :])
            rdma = pltpu.make_async_remote_copy(
                src_ref=src,
                dst_ref=rs_recv.at[s, rows, :],
                send_sem=send_sems.at[_RS + 2 * s + h],
                recv_sem=recv_sems.at[_RS + 2 * s + h],
                device_id=right,
                device_id_type=pl.DeviceIdType.MESH,
            )
            rdma.start()
            return rdma

        rs = [[None, None] for _ in range(NZ - 1)]
        send0[0:HC, :] = pc[0, 0:HC, :].astype(BF16)
        rs[0][0] = mk_rs(0, 0)
        send0[HC:CHUNK, :] = pc[0, HC:CHUNK, :].astype(BF16)
        rs[0][1] = mk_rs(0, 1)
        l2 = pchunk_copy((r + NZ - 3) % NZ, 0)
        l1.wait()
        l3 = None
        for s in range(NZ - 2):
            for h in range(2):
                with jax.named_scope(f"rs#s={s}_h={h}"):
                    if s == 1 and h == 0:
                        l2.wait()
                    rs[s][h].wait()
                    slot = 1 - s
                    rows = slice(h * HC, (h + 1) * HC)
                    acc = (rs_recv[s, rows, :].astype(F32)
                           + pc[slot, rows, :])
                    rs_recv[s, rows, :] = acc.astype(BF16)
                    rs[s + 1][h] = mk_rs(s + 1, h)
                    if s == 0 and h == 1:
                        l3 = pchunk_copy(r, 1)

        l3.wait()
        cr.wait()

        def ln_half(h):
            rows = slice(h * HC, (h + 1) * HC)
            yv = (rs_recv[NZ - 2, rows, :].astype(F32) + pc[1, rows, :]
                  + resid_chunk[rows, :])
            ms = jnp.mean(yv * yv, axis=-1, keepdims=True)
            outv = yv * lax.rsqrt(ms + EPS) * gamma_ref[:, :]
            q_buf[pl.ds(r * CHUNK + h * HC, HC), :] = outv.astype(BF16)

        for h in range(2):
            with jax.named_scope(f"ln#h={h}"):
                rs[NZ - 2][h].wait()
                ln_half(h)

        stores = []

        def store(src_slice, row_start):
            cp = pltpu.make_async_copy(
                src_slice, out_ref.at[pl.ds(row_start, CHUNK), :],
                store_sems.at[len(stores)])
            cp.start()
            stores.append(cp)

        def abs_k(t):
            return (r + NZ - t) % NZ

        ax_list, ay_list = [], []

        def start_a(t):
            k = abs_k(t)
            sl = (pl.ds(k * CHUNK, CHUNK), slice(None))
            a = pltpu.make_async_remote_copy(
                src_ref=q_buf.at[sl], dst_ref=qx_recv.at[sl],
                send_sem=send_sems.at[_AX + t],
                recv_sem=recv_sems.at[_AX + t],
                device_id=xn, device_id_type=pl.DeviceIdType.MESH,
            )
            a.start()
            ax_list.append(a)
            a = pltpu.make_async_remote_copy(
                src_ref=q_buf.at[sl], dst_ref=qy_recv.at[sl],
                send_sem=send_sems.at[_AY + t],
                recv_sem=recv_sems.at[_AY + t],
                device_id=yn, device_id_type=pl.DeviceIdType.MESH,
            )
            a.start()
            ay_list.append(a)

        def mk_ag(t):
            sl = (pl.ds(abs_k(t) * CHUNK, CHUNK), slice(None))
            rdma = pltpu.make_async_remote_copy(
                src_ref=q_buf.at[sl], dst_ref=q_buf.at[sl],
                send_sem=send_sems.at[_AG + t],
                recv_sem=recv_sems.at[_AG + t],
                device_id=right, device_id_type=pl.DeviceIdType.MESH,
            )
            rdma.start()
            return rdma

        do_ag = _PHASES in ("ag", "full")
        do_xy = _PHASES == "full"
        if not do_xy:
            start_a = lambda t: None

        if do_ag:
            ag = mk_ag(0)
        start_a(0)
        store(q_buf.at[pl.ds(r * CHUNK, CHUNK), :], base + r * CHUNK)
        for t in range(NZ - 1) if do_ag else ():
            with jax.named_scope(f"ag#t={t}"):
                ag.wait()
                if t < NZ - 2:
                    ag = mk_ag(t + 1)
                k_in = abs_k(t + 1)
                start_a(t + 1)
                store(q_buf.at[pl.ds(k_in * CHUNK, CHUNK), :],
                      base + k_in * CHUNK)

        qx = 2 * (1 - x) + y
        qy = 2 * x + (1 - y)
        qd = 2 * (1 - x) + (1 - y)
        b_list = []
        for t in range(NZ) if do_xy else ():
          with jax.named_scope(f"xy#t={t}"):
            k = abs_k(t)
            sl = (pl.ds(k * CHUNK, CHUNK), slice(None))
            if t < 2:
                ay_list[t].wait()
                b = pltpu.make_async_remote_copy(
                    src_ref=qy_recv.at[sl], dst_ref=qd_buf.at[sl],
                    send_sem=send_sems.at[_BX + t],
                    recv_sem=recv_sems.at[_BX + t],
                    device_id=xn, device_id_type=pl.DeviceIdType.MESH,
                )
                b.start()
                ax_list[t].wait()
            else:
                ax_list[t].wait()
                b = pltpu.make_async_remote_copy(
                    src_ref=qx_recv.at[sl], dst_ref=qd_buf.at[sl],
                    send_sem=send_sems.at[_BY + (t - 2)],
                    recv_sem=recv_sems.at[_BY + (t - 2)],
                    device_id=yn, device_id_type=pl.DeviceIdType.MESH,
                )
                b.start()
                ay_list[t].wait()
            b_list.append(b)
            store(qx_recv.at[sl], qx * QROWS + k * CHUNK)
            store(qy_recv.at[sl], qy * QROWS + k * CHUNK)

        for t in range(NZ) if do_xy else ():
            with jax.named_scope(f"brelay#t={t}"):
                b_list[t].wait()
                k = abs_k(t)
                store(qd_buf.at[pl.ds(k * CHUNK, CHUNK), :],
                      qd * QROWS + k * CHUNK)

        with jax.named_scope("drain"):
            for cp in stores:
                cp.wait()

    out_shape = jax.ShapeDtypeStruct((M, D), BF16)
    return pl.pallas_call(
        body,
        out_shape=out_shape,
        in_specs=[
            pl.BlockSpec(memory_space=pl.ANY),
            pl.BlockSpec(memory_space=pl.ANY),
            pl.BlockSpec(memory_space=pltpu.VMEM),
        ],
        out_specs=pl.BlockSpec(memory_space=pl.ANY),
        scratch_shapes=[
            pltpu.VMEM((2, CHUNK, D), F32),
            pltpu.VMEM((CHUNK, D), BF16),
            pltpu.VMEM((NZ - 1, CHUNK, D), BF16),
            pltpu.VMEM((QROWS, D), BF16),
            pltpu.VMEM((QROWS, D), BF16),
            pltpu.VMEM((QROWS, D), BF16),
            pltpu.VMEM((QROWS, D), BF16),
            pltpu.VMEM((CHUNK, D), F32),
            pltpu.SemaphoreType.DMA((21,)),
            pltpu.SemaphoreType.DMA((21,)),
            pltpu.SemaphoreType.DMA((3,)),
            pltpu.SemaphoreType.DMA((16,)),
        ],
        compiler_params=pltpu.CompilerParams(
            collective_id=0,
            vmem_limit_bytes=100 * 1024 * 1024,
        ),
    )(partial, resid, gamma2)


# device time: 234025 ns/iter; 1.0516x vs baseline; 1.0477x over previous
import os

import jax
import jax.numpy as jnp
from jax import lax
from jax.experimental import pallas as pl
from jax.experimental.pallas import tpu as pltpu

_PHASES = os.environ.get("SCB_KERNEL_PHASES", "full")

NZ = 4
M = 8192
D = 2048
QROWS = M // 4
CHUNK = QROWS // NZ
EPS = 1e-6
F32 = jnp.float32
BF16 = jnp.bfloat16

HC = CHUNK // 2

_RS = 0
_AG = 6
_AX = 9
_AY = 13
_BX = 17
_BY = 19
_SW = 21


def kernel(partial, resid, gamma):
    gamma2 = gamma.reshape(1, D)

    def body(part_ref, resid_ref, gamma_ref, out_ref,
             pc, send0, rs_recv, q_buf, qx_recv, qy_recv, qd_buf,
             resid_chunk, send_sems, recv_sems, local_sems, store_sems):
        x = lax.axis_index("x")
        y = lax.axis_index("y")
        r = lax.axis_index("z")
        right = (x, y, (r + 1) % NZ)
        left = (x, y, (r + NZ - 1) % NZ)
        xn = (1 - x, y, r)
        yn = (x, 1 - y, r)
        par = r % 2
        pz = (x, y, r + 1 - 2 * par)

        q = 2 * x + y
        base = q * QROWS

        barrier = pltpu.get_barrier_semaphore()
        for nbr in (left, right, xn, yn):
            pl.semaphore_signal(
                barrier, inc=1, device_id=nbr,
                device_id_type=pl.DeviceIdType.MESH,
            )
        pl.semaphore_wait(barrier, 4)

        def pchunk_copy(c, slot):
            cp = pltpu.make_async_copy(
                part_ref.at[0, pl.ds(base + c * CHUNK, CHUNK), :],
                pc.at[slot], local_sems.at[slot])
            cp.start()
            return cp

        l0 = pchunk_copy((r + NZ - 1) % NZ, 0)
        l1 = pchunk_copy((r + NZ - 2) % NZ, 1)
        cr = pltpu.make_async_copy(
            resid_ref.at[pl.ds(base + r * CHUNK, CHUNK), :],
            resid_chunk, local_sems.at[2])
        cr.start()

        l0.wait()

        def mk_rs(s, h):
            rows = pl.ds(h * HC, HC)
            src = (send0.at[rows, :] if s == 0
                   else rs_recv.at[s - 1, rows, :])
            rdma = pltpu.make_async_remote_copy(
                src_ref=src,
                dst_ref=rs_recv.at[s, rows, :],
                send_sem=send_sems.at[_RS + 2 * s + h],
                recv_sem=recv_sems.at[_RS + 2 * s + h],
                device_id=right,
                device_id_type=pl.DeviceIdType.MESH,
            )
            rdma.start()
            return rdma

        rs = [[None, None] for _ in range(NZ - 1)]
        send0[0:HC, :] = pc[0, 0:HC, :].astype(BF16)
        rs[0][0] = mk_rs(0, 0)
        send0[HC:CHUNK, :] = pc[0, HC:CHUNK, :].astype(BF16)
        rs[0][1] = mk_rs(0, 1)
        l2 = pchunk_copy((r + NZ - 3) % NZ, 0)
        l1.wait()
        l3 = None
        for s in range(NZ - 2):
            for h in range(2):
                with jax.named_scope(f"rs#s={s}_h={h}"):
                    if s == 1 and h == 0:
                        l2.wait()
                    rs[s][h].wait()
                    slot = 1 - s
                    rows = slice(h * HC, (h + 1) * HC)
                    acc = (rs_recv[s, rows, :].astype(F32)
                           + pc[slot, rows, :])
                    rs_recv[s, rows, :] = acc.astype(BF16)
                    rs[s + 1][h] = mk_rs(s + 1, h)
                    if s == 0 and h == 1:
                        l3 = pchunk_copy(r, 1)

        l3.wait()
        cr.wait()

        def ln_half(h):
            rows = slice(h * HC, (h + 1) * HC)
            yv = (rs_recv[NZ - 2, rows, :].astype(F32) + pc[1, rows, :]
                  + resid_chunk[rows, :])
            ms = jnp.mean(yv * yv, axis=-1, keepdims=True)
            outv = yv * lax.rsqrt(ms + EPS) * gamma_ref[:, :]
            q_buf[pl.ds(r * CHUNK + h * HC, HC), :] = outv.astype(BF16)

        for h in range(2):
            with jax.named_scope(f"ln#h={h}"):
                rs[NZ - 2][h].wait()
                ln_half(h)

        stores = []

        def store(src_slice, row_start):
            cp = pltpu.make_async_copy(
                src_slice, out_ref.at[pl.ds(row_start, CHUNK), :],
                store_sems.at[len(stores)])
            cp.start()
            stores.append(cp)

        def abs_k(t):
            return (r + NZ - t) % NZ

        ax_list, ay_list = [], []

        def start_a(t):
            k = abs_k(t)
            if t == NZ - 1:
                sl = (pl.ds(k * CHUNK + par * HC, HC), slice(None))
            else:
                sl = (pl.ds(k * CHUNK, CHUNK), slice(None))
            a = pltpu.make_async_remote_copy(
                src_ref=q_buf.at[sl], dst_ref=qx_recv.at[sl],
                send_sem=send_sems.at[_AX + t],
                recv_sem=recv_sems.at[_AX + t],
                device_id=xn, device_id_type=pl.DeviceIdType.MESH,
            )
            a.start()
            ax_list.append(a)
            a = pltpu.make_async_remote_copy(
                src_ref=q_buf.at[sl], dst_ref=qy_recv.at[sl],
                send_sem=send_sems.at[_AY + t],
                recv_sem=recv_sems.at[_AY + t],
                device_id=yn, device_id_type=pl.DeviceIdType.MESH,
            )
            a.start()
            ay_list.append(a)

        def mk_ag(t):
            sl = (pl.ds(abs_k(t) * CHUNK, CHUNK), slice(None))
            rdma = pltpu.make_async_remote_copy(
                src_ref=q_buf.at[sl], dst_ref=q_buf.at[sl],
                send_sem=send_sems.at[_AG + t],
                recv_sem=recv_sems.at[_AG + t],
                device_id=right, device_id_type=pl.DeviceIdType.MESH,
            )
            rdma.start()
            return rdma

        do_ag = _PHASES in ("ag", "full")
        do_xy = _PHASES == "full"
        if not do_xy:
            start_a = lambda t: None

        if do_ag:
            ag = mk_ag(0)
        start_a(0)
        store(q_buf.at[pl.ds(r * CHUNK, CHUNK), :], base + r * CHUNK)
        for t in range(NZ - 1) if do_ag else ():
            with jax.named_scope(f"ag#t={t}"):
                ag.wait()
                if t < NZ - 2:
                    ag = mk_ag(t + 1)
                k_in = abs_k(t + 1)
                start_a(t + 1)
                store(q_buf.at[pl.ds(k_in * CHUNK, CHUNK), :],
                      base + k_in * CHUNK)

        qx = 2 * (1 - x) + y
        qy = 2 * x + (1 - y)
        qd = 2 * (1 - x) + (1 - y)
        b_list = []
        for t in range(NZ) if do_xy else ():
          with jax.named_scope(f"xy#t={t}"):
            k = abs_k(t)
            sl = (pl.ds(k * CHUNK, CHUNK), slice(None))
            if t < 2:
                ay_list[t].wait()
                b = pltpu.make_async_remote_copy(
                    src_ref=qy_recv.at[sl], dst_ref=qd_buf.at[sl],
                    send_sem=send_sems.at[_BX + t],
                    recv_sem=recv_sems.at[_BX + t],
                    device_id=xn, device_id_type=pl.DeviceIdType.MESH,
                )
                b.start()
                ax_list[t].wait()
            elif t == 2:
                ax_list[t].wait()
                cs = (r + 2 - 2 * par) % NZ
                sw_sl = (pl.ds(cs * CHUNK + par * HC, HC), slice(None))

                def mk_swap(buf, idx):
                    rdma = pltpu.make_async_remote_copy(
                        src_ref=buf.at[sw_sl], dst_ref=buf.at[sw_sl],
                        send_sem=send_sems.at[_SW + idx],
                        recv_sem=recv_sems.at[_SW + idx],
                        device_id=pz,
                        device_id_type=pl.DeviceIdType.MESH,
                    )
                    rdma.start()
                    return rdma

                swap_x = mk_swap(qx_recv, 0)
                b = pltpu.make_async_remote_copy(
                    src_ref=qx_recv.at[sl], dst_ref=qd_buf.at[sl],
                    send_sem=send_sems.at[_BY],
                    recv_sem=recv_sems.at[_BY],
                    device_id=yn, device_id_type=pl.DeviceIdType.MESH,
                )
                b.start()
                ay_list[t].wait()
                swap_y = mk_swap(qy_recv, 1)
            else:
                ax_list[t].wait()
                swap_x.wait()
                b = pltpu.make_async_remote_copy(
                    src_ref=qx_recv.at[sl], dst_ref=qd_buf.at[sl],
                    send_sem=send_sems.at[_BY + 1],
                    recv_sem=recv_sems.at[_BY + 1],
                    device_id=yn, device_id_type=pl.DeviceIdType.MESH,
                )
                b.start()
                store(qx_recv.at[sl], qx * QROWS + k * CHUNK)
                ay_list[t].wait()
                swap_y.wait()
            b_list.append(b)
            if t < NZ - 1:
                store(qx_recv.at[sl], qx * QROWS + k * CHUNK)
            store(qy_recv.at[sl], qy * QROWS + k * CHUNK)

        for t in range(NZ) if do_xy else ():
            with jax.named_scope(f"brelay#t={t}"):
                b_list[t].wait()
                k = abs_k(t)
                store(qd_buf.at[pl.ds(k * CHUNK, CHUNK), :],
                      qd * QROWS + k * CHUNK)

        with jax.named_scope("drain"):
            for cp in stores:
                cp.wait()

    out_shape = jax.ShapeDtypeStruct((M, D), BF16)
    return pl.pallas_call(
        body,
        out_shape=out_shape,
        in_specs=[
            pl.BlockSpec(memory_space=pl.ANY),
            pl.BlockSpec(memory_space=pl.ANY),
            pl.BlockSpec(memory_space=pltpu.VMEM),
        ],
        out_specs=pl.BlockSpec(memory_space=pl.ANY),
        scratch_shapes=[
            pltpu.VMEM((2, CHUNK, D), F32),
            pltpu.VMEM((CHUNK, D), BF16),
            pltpu.VMEM((NZ - 1, CHUNK, D), BF16),
            pltpu.VMEM((QROWS, D), BF16),
            pltpu.VMEM((QROWS, D), BF16),
            pltpu.VMEM((QROWS, D), BF16),
            pltpu.VMEM((QROWS, D), BF16),
            pltpu.VMEM((CHUNK, D), F32),
            pltpu.SemaphoreType.DMA((23,)),
            pltpu.SemaphoreType.DMA((23,)),
            pltpu.SemaphoreType.DMA((3,)),
            pltpu.SemaphoreType.DMA((16,)),
        ],
        compiler_params=pltpu.CompilerParams(
            collective_id=0,
            vmem_limit_bytes=100 * 1024 * 1024,
        ),
    )(partial, resid, gamma2)
